# Initial kernel scaffold; baseline (speedup 1.0000x reference)
import jax
import jax.numpy as jnp
from jax.experimental import pallas as pl


def kernel(z, cheb):
    raise NotImplementedError("write your pallas kernel here")


# speedup vs baseline: 1.1627x; 1.1627x over previous
import functools

import jax
import jax.numpy as jnp
from jax import lax
from jax.experimental import pallas as pl
from jax.experimental.pallas import tpu as pltpu
from jax.experimental.pallas import tpu_sc as plsc

_X = 1000000
_YC = 32
_N = 819200
_LO = -1.0

_NC, _NS, _L = 2, 16, 16
_NW = _NC * _NS
_QW = _N // _NW
_CHUNK = 1024
_NCHUNK = _QW // _CHUNK
_BQ = 128
_NB = _CHUNK // _BQ
_NG = _BQ // _L


def _series_eval(z, cheb):
    mesh = plsc.VectorSubcoreMesh(core_axis_name="c", subcore_axis_name="s")

    @functools.partial(
        pl.kernel,
        out_type=jax.ShapeDtypeStruct((_N,), jnp.float32),
        mesh=mesh,
        compiler_params=pltpu.CompilerParams(
            needs_layout_passes=False, use_tc_tiling_on_sc=False),
        scratch_types=[
            pltpu.VMEM((_CHUNK,), jnp.float32),
            pltpu.VMEM((_NB, _BQ), jnp.int32),
            pltpu.VMEM((_CHUNK,), jnp.float32),
            pltpu.VMEM((_NB, _BQ, _YC), jnp.float32),
            pltpu.VMEM((_CHUNK,), jnp.float32),
            pltpu.SemaphoreType.DMA,
        ],
    )
    def run(z_hbm, cheb_hbm, out_hbm, z_v, idx_v, y_v, rows_v, out_v, sem):
        wid = lax.axis_index("s") * _NC + lax.axis_index("c")
        base = wid * _QW

        def chunk_body(c, carry):
            off = base + c * _CHUNK
            pltpu.sync_copy(z_hbm.at[pl.ds(off, _CHUNK)], z_v)

            for i in range(_CHUNK // _L):
                t = z_v[pl.ds(i * _L, _L)] - _LO
                xi = (t * 0.5).astype(jnp.int32)
                xi = jnp.minimum(xi, _X - 1)
                y = t - 2.0 * xi.astype(jnp.float32) + _LO
                y = jnp.minimum(jnp.maximum(y, -1.0 + 1e-6), 1.0 - 1e-6)
                idx_v[i // _NG, pl.ds((i % _NG) * _L, _L)] = xi
                y_v[pl.ds(i * _L, _L)] = y

            copies = [
                pltpu.async_copy(cheb_hbm.at[idx_v.at[b]], rows_v.at[b], sem)
                for b in range(_NB)
            ]
            for cp in copies:
                cp.wait()

            for b in range(_NB):
                rows_b = rows_v.at[b]

                def group_body(g, _, b=b, rows_b=rows_b):
                    q0 = b * _BQ + g * _L
                    qidx = lax.iota(jnp.int32, _L) + g * _L
                    y = y_v[pl.ds(q0, _L)]
                    y2 = y + y
                    bk1 = plsc.load_gather(
                        rows_b, [qidx, jnp.full((_L,), _YC - 1, jnp.int32)])
                    bk2 = jnp.zeros((_L,), jnp.float32)
                    for n in range(_YC - 2, 0, -1):
                        a = plsc.load_gather(
                            rows_b, [qidx, jnp.full((_L,), n, jnp.int32)])
                        bk1, bk2 = a + y2 * bk1 - bk2, bk1
                    a0 = plsc.load_gather(
                        rows_b, [qidx, jnp.full((_L,), 0, jnp.int32)])
                    out_v[pl.ds(q0, _L)] = a0 + y * bk1 - bk2
                    return _

                lax.fori_loop(0, _NG, group_body, 0)

            pltpu.sync_copy(out_v, out_hbm.at[pl.ds(off, _CHUNK)])
            return carry

        lax.fori_loop(0, _NCHUNK, chunk_body, 0)

    return run(z, cheb)


def kernel(z, cheb):
    return _series_eval(z, cheb)
